# baseline (speedup 1.0000x reference)
import functools

import jax
import jax.numpy as jnp
import numpy as np
from jax import lax
from jax.experimental import pallas as pl
from jax.experimental.pallas import tpu as pltpu
from jax.experimental.pallas import tpu_sc as plsc

_OUTPUT_SIZE = 1024


def _choose_indices(B, L):
    key = jax.random.key(42)
    noise = jax.random.uniform(jax.random.fold_in(key, 1), (B, L))
    indices = jnp.argsort(noise, axis=-1)[:, :_OUTPUT_SIZE]
    return jnp.sort(indices, axis=-1)


@functools.lru_cache(maxsize=None)
def _flat_indices_const(B, L):
    try:
        with jax.ensure_compile_time_eval():
            idx = _choose_indices(B, L)
            flat = (idx + jnp.arange(B)[:, None] * L).reshape(-1)
            return np.asarray(flat, dtype=np.int32)
    except Exception:
        return None


@functools.lru_cache(maxsize=None)
def _make_gather(V, D, B):
    info = plsc.get_sparse_core_info()
    NC, NS = info.num_cores, info.num_subcores
    NW = NC * NS
    assert B % NW == 0 and (B // NW) % 8 == 0
    b_per_w = B // NW
    chunk = min(16, b_per_w)
    n_chunks = b_per_w // chunk
    assert n_chunks % 2 == 0 and n_chunks >= 4
    mesh = plsc.VectorSubcoreMesh(core_axis_name="c", subcore_axis_name="s")

    @functools.partial(
        pl.kernel,
        mesh=mesh,
        out_type=jax.ShapeDtypeStruct((B, D), jnp.float32),
        scratch_types=[
            pltpu.VMEM((b_per_w,), jnp.int32),
            pltpu.VMEM((chunk, D), jnp.float32),
            pltpu.VMEM((chunk, D), jnp.float32),
            pltpu.SemaphoreType.DMA,
            pltpu.SemaphoreType.DMA,
            pltpu.SemaphoreType.DMA,
            pltpu.SemaphoreType.DMA,
        ],
    )
    def gather_kernel(table_hbm, idx_hbm, out_hbm, idx_v,
                      buf0, buf1, gsem0, gsem1, wsem0, wsem1):
        bufs = (buf0, buf1)
        gsems = (gsem0, gsem1)
        wsems = (wsem0, wsem1)
        wid = lax.axis_index("s") * NC + lax.axis_index("c")
        base = wid * b_per_w
        pltpu.sync_copy(idx_hbm.at[pl.ds(base, b_per_w)], idx_v)

        def gather_start(c, b):
            pltpu.async_copy(
                table_hbm.at[idx_v.at[pl.ds(c * chunk, chunk)]],
                bufs[b], gsems[b])

        def gather_wait(b):
            pltpu.make_async_copy(
                table_hbm.at[idx_v.at[pl.ds(0, chunk)]],
                bufs[b], gsems[b]).wait()

        def write_start(c, b):
            pltpu.async_copy(
                bufs[b], out_hbm.at[pl.ds(base + c * chunk, chunk)], wsems[b])

        def write_wait(b):
            pltpu.make_async_copy(
                bufs[b], out_hbm.at[pl.ds(base, chunk)], wsems[b]).wait()

        gather_start(0, 0)
        gather_wait(0)
        write_start(0, 0)
        gather_start(1, 1)

        def body(g, carry):
            c0 = 2 * g + 1
            gather_wait(1)
            write_start(c0, 1)
            write_wait(0)
            gather_start(c0 + 1, 0)
            gather_wait(0)
            write_start(c0 + 1, 0)
            write_wait(1)
            gather_start(c0 + 2, 1)
            return carry

        lax.fori_loop(0, (n_chunks - 2) // 2, body, 0)
        gather_wait(1)
        write_start(n_chunks - 1, 1)
        write_wait(0)
        write_wait(1)

    return gather_kernel


def kernel(inputs):
    B, L, D = inputs.shape
    const = _flat_indices_const(B, L)
    if const is not None:
        flat_idx = jnp.asarray(const)
    else:
        indices = _choose_indices(B, L)
        flat_idx = (indices + jnp.arange(B)[:, None] * L).reshape(-1).astype(jnp.int32)
    table = inputs.reshape(B * L, D)
    out = _make_gather(B * L, D, B * _OUTPUT_SIZE)(table, flat_idx)
    return out.reshape(B, _OUTPUT_SIZE, D)


# speedup vs baseline: 1.0199x; 1.0199x over previous
import functools

import jax
import jax.numpy as jnp
import numpy as np
from jax import lax
from jax.experimental import pallas as pl
from jax.experimental.pallas import tpu as pltpu
from jax.experimental.pallas import tpu_sc as plsc

_OUTPUT_SIZE = 1024


def _choose_indices(B, L):
    key = jax.random.key(42)
    noise = jax.random.uniform(jax.random.fold_in(key, 1), (B, L))
    indices = jnp.argsort(noise, axis=-1)[:, :_OUTPUT_SIZE]
    return jnp.sort(indices, axis=-1)


@functools.lru_cache(maxsize=None)
def _flat_indices_const(B, L):
    try:
        with jax.ensure_compile_time_eval():
            idx = _choose_indices(B, L)
            flat = (idx + jnp.arange(B)[:, None] * L).reshape(-1)
            return np.asarray(flat, dtype=np.int32)
    except Exception:
        return None


@functools.lru_cache(maxsize=None)
def _make_gather(V, D, B):
    info = plsc.get_sparse_core_info()
    NC, NS = info.num_cores, info.num_subcores
    NW = NC * NS
    assert B % NW == 0 and (B // NW) % 8 == 0
    b_per_w = B // NW
    chunk = min(16, b_per_w)
    n_chunks = b_per_w // chunk
    assert n_chunks % 2 == 0 and n_chunks >= 4
    mesh = plsc.VectorSubcoreMesh(core_axis_name="c", subcore_axis_name="s")

    @functools.partial(
        pl.kernel,
        mesh=mesh,
        out_type=jax.ShapeDtypeStruct((B, D), jnp.float32),
        scratch_types=[
            pltpu.VMEM((b_per_w,), jnp.int32),
            pltpu.VMEM((chunk, D), jnp.float32),
            pltpu.VMEM((chunk, D), jnp.float32),
            pltpu.VMEM((chunk, D), jnp.float32),
            pltpu.SemaphoreType.DMA,
            pltpu.SemaphoreType.DMA,
            pltpu.SemaphoreType.DMA,
            pltpu.SemaphoreType.DMA,
            pltpu.SemaphoreType.DMA,
            pltpu.SemaphoreType.DMA,
        ],
    )
    def gather_kernel(table_hbm, idx_hbm, out_hbm, idx_v,
                      buf0, buf1, buf2, gsem0, gsem1, gsem2,
                      wsem0, wsem1, wsem2):
        bufs = (buf0, buf1, buf2)
        gsems = (gsem0, gsem1, gsem2)
        wsems = (wsem0, wsem1, wsem2)
        wid = lax.axis_index("s") * NC + lax.axis_index("c")
        base = wid * b_per_w
        pltpu.sync_copy(idx_hbm.at[pl.ds(base, b_per_w)], idx_v)

        def gather_start(c, b):
            pltpu.async_copy(
                table_hbm.at[idx_v.at[pl.ds(c * chunk, chunk)]],
                bufs[b], gsems[b])

        def gather_wait(b):
            pltpu.make_async_copy(
                table_hbm.at[idx_v.at[pl.ds(0, chunk)]],
                bufs[b], gsems[b]).wait()

        def write_start(c, b):
            pltpu.async_copy(
                bufs[b], out_hbm.at[pl.ds(base + c * chunk, chunk)], wsems[b])

        def write_wait(b):
            pltpu.make_async_copy(
                bufs[b], out_hbm.at[pl.ds(base, chunk)], wsems[b]).wait()

        gather_start(0, 0)
        gather_start(1, 1)
        for c in range(n_chunks):
            b = c % 3
            gather_wait(b)
            write_start(c, b)
            nxt = c + 2
            if nxt < n_chunks:
                if nxt >= 3:
                    write_wait(nxt % 3)
                gather_start(nxt, nxt % 3)
        for c in range(n_chunks - 3, n_chunks):
            write_wait(c % 3)

    return gather_kernel


def kernel(inputs):
    B, L, D = inputs.shape
    const = _flat_indices_const(B, L)
    if const is not None:
        flat_idx = jnp.asarray(const)
    else:
        indices = _choose_indices(B, L)
        flat_idx = (indices + jnp.arange(B)[:, None] * L).reshape(-1).astype(jnp.int32)
    table = inputs.reshape(B * L, D)
    out = _make_gather(B * L, D, B * _OUTPUT_SIZE)(table, flat_idx)
    return out.reshape(B, _OUTPUT_SIZE, D)


# speedup vs baseline: 1.0304x; 1.0104x over previous
import functools

import jax
import jax.numpy as jnp
import numpy as np
from jax import lax
from jax.experimental import pallas as pl
from jax.experimental.pallas import tpu as pltpu
from jax.experimental.pallas import tpu_sc as plsc

_OUTPUT_SIZE = 1024


def _choose_indices(B, L):
    key = jax.random.key(42)
    noise = jax.random.uniform(jax.random.fold_in(key, 1), (B, L))
    indices = jnp.argsort(noise, axis=-1)[:, :_OUTPUT_SIZE]
    return jnp.sort(indices, axis=-1)


@functools.lru_cache(maxsize=None)
def _flat_indices_const(B, L):
    try:
        with jax.ensure_compile_time_eval():
            idx = _choose_indices(B, L)
            flat = (idx + jnp.arange(B)[:, None] * L).reshape(-1)
            return np.asarray(flat, dtype=np.int32)
    except Exception:
        return None


@functools.lru_cache(maxsize=None)
def _make_gather(V, D, B):
    info = plsc.get_sparse_core_info()
    NC, NS = info.num_cores, info.num_subcores
    NW = NC * NS
    assert B % NW == 0 and (B // NW) % 8 == 0
    b_per_w = B // NW
    chunk = min(8, b_per_w)
    n_chunks = b_per_w // chunk
    assert n_chunks % 2 == 0 and n_chunks >= 4
    mesh = plsc.VectorSubcoreMesh(core_axis_name="c", subcore_axis_name="s")

    @functools.partial(
        pl.kernel,
        mesh=mesh,
        out_type=jax.ShapeDtypeStruct((B, D), jnp.float32),
        scratch_types=[
            pltpu.VMEM((b_per_w,), jnp.int32),
            pltpu.VMEM((chunk, D), jnp.float32),
            pltpu.VMEM((chunk, D), jnp.float32),
            pltpu.VMEM((chunk, D), jnp.float32),
            pltpu.SemaphoreType.DMA,
            pltpu.SemaphoreType.DMA,
            pltpu.SemaphoreType.DMA,
            pltpu.SemaphoreType.DMA,
            pltpu.SemaphoreType.DMA,
            pltpu.SemaphoreType.DMA,
        ],
    )
    def gather_kernel(table_hbm, idx_hbm, out_hbm, idx_v,
                      buf0, buf1, buf2, gsem0, gsem1, gsem2,
                      wsem0, wsem1, wsem2):
        bufs = (buf0, buf1, buf2)
        gsems = (gsem0, gsem1, gsem2)
        wsems = (wsem0, wsem1, wsem2)
        wid = lax.axis_index("s") * NC + lax.axis_index("c")
        base = wid * b_per_w
        pltpu.sync_copy(idx_hbm.at[pl.ds(base, b_per_w)], idx_v)

        def gather_start(c, b):
            pltpu.async_copy(
                table_hbm.at[idx_v.at[pl.ds(c * chunk, chunk)]],
                bufs[b], gsems[b])

        def gather_wait(b):
            pltpu.make_async_copy(
                table_hbm.at[idx_v.at[pl.ds(0, chunk)]],
                bufs[b], gsems[b]).wait()

        def write_start(c, b):
            pltpu.async_copy(
                bufs[b], out_hbm.at[pl.ds(base + c * chunk, chunk)], wsems[b])

        def write_wait(b):
            pltpu.make_async_copy(
                bufs[b], out_hbm.at[pl.ds(base, chunk)], wsems[b]).wait()

        gather_start(0, 0)
        gather_start(1, 1)
        for c in range(n_chunks):
            b = c % 3
            gather_wait(b)
            write_start(c, b)
            nxt = c + 2
            if nxt < n_chunks:
                if nxt >= 3:
                    write_wait(nxt % 3)
                gather_start(nxt, nxt % 3)
        for c in range(n_chunks - 3, n_chunks):
            write_wait(c % 3)

    return gather_kernel


def kernel(inputs):
    B, L, D = inputs.shape
    const = _flat_indices_const(B, L)
    if const is not None:
        flat_idx = jnp.asarray(const)
    else:
        indices = _choose_indices(B, L)
        flat_idx = (indices + jnp.arange(B)[:, None] * L).reshape(-1).astype(jnp.int32)
    table = inputs.reshape(B * L, D)
    out = _make_gather(B * L, D, B * _OUTPUT_SIZE)(table, flat_idx)
    return out.reshape(B, _OUTPUT_SIZE, D)


# speedup vs baseline: 1.0383x; 1.0077x over previous
import functools

import jax
import jax.numpy as jnp
import numpy as np
from jax import lax
from jax.experimental import pallas as pl
from jax.experimental.pallas import tpu as pltpu
from jax.experimental.pallas import tpu_sc as plsc

_OUTPUT_SIZE = 1024


def _choose_indices(B, L):
    key = jax.random.key(42)
    noise = jax.random.uniform(jax.random.fold_in(key, 1), (B, L))
    indices = jnp.argsort(noise, axis=-1)[:, :_OUTPUT_SIZE]
    return jnp.sort(indices, axis=-1)


@functools.lru_cache(maxsize=None)
def _flat_indices_const(B, L):
    try:
        with jax.ensure_compile_time_eval():
            idx = _choose_indices(B, L)
            flat = (idx + jnp.arange(B)[:, None] * L).reshape(-1)
            return np.asarray(flat, dtype=np.int32)
    except Exception:
        return None


@functools.lru_cache(maxsize=None)
def _make_gather(V, D, B):
    info = plsc.get_sparse_core_info()
    NC, NS = info.num_cores, info.num_subcores
    NW = NC * NS
    assert B % NW == 0 and (B // NW) % 8 == 0
    b_per_w = B // NW
    chunk = min(8, b_per_w)
    n_chunks = b_per_w // chunk
    assert n_chunks % 2 == 0 and n_chunks >= 4
    mesh = plsc.VectorSubcoreMesh(core_axis_name="c", subcore_axis_name="s")

    @functools.partial(
        pl.kernel,
        mesh=mesh,
        out_type=jax.ShapeDtypeStruct((B, D), jnp.float32),
        scratch_types=[
            pltpu.VMEM((b_per_w,), jnp.int32),
        ]
        + [pltpu.VMEM((chunk, D), jnp.float32) for _ in range(4)]
        + [pltpu.SemaphoreType.DMA for _ in range(8)],
    )
    def gather_kernel(table_hbm, idx_hbm, out_hbm, idx_v, *scratch):
        bufs = scratch[:4]
        gsems = scratch[4:8]
        wsems = scratch[8:]
        wid = lax.axis_index("s") * NC + lax.axis_index("c")
        base = wid * b_per_w
        pltpu.sync_copy(idx_hbm.at[pl.ds(base, b_per_w)], idx_v)

        def gather_start(c, b):
            pltpu.async_copy(
                table_hbm.at[idx_v.at[pl.ds(c * chunk, chunk)]],
                bufs[b], gsems[b])

        def gather_wait(b):
            pltpu.make_async_copy(
                table_hbm.at[idx_v.at[pl.ds(0, chunk)]],
                bufs[b], gsems[b]).wait()

        def write_start(c, b):
            pltpu.async_copy(
                bufs[b], out_hbm.at[pl.ds(base + c * chunk, chunk)], wsems[b])

        def write_wait(b):
            pltpu.make_async_copy(
                bufs[b], out_hbm.at[pl.ds(base, chunk)], wsems[b]).wait()

        gather_start(0, 0)
        gather_start(1, 1)
        gather_start(2, 2)
        for c in range(n_chunks):
            b = c % 4
            gather_wait(b)
            write_start(c, b)
            nxt = c + 3
            if nxt < n_chunks:
                if nxt >= 4:
                    write_wait(nxt % 4)
                gather_start(nxt, nxt % 4)
        for c in range(n_chunks - 4, n_chunks):
            write_wait(c % 4)

    return gather_kernel


def kernel(inputs):
    B, L, D = inputs.shape
    const = _flat_indices_const(B, L)
    if const is not None:
        flat_idx = jnp.asarray(const)
    else:
        indices = _choose_indices(B, L)
        flat_idx = (indices + jnp.arange(B)[:, None] * L).reshape(-1).astype(jnp.int32)
    table = inputs.reshape(B * L, D)
    out = _make_gather(B * L, D, B * _OUTPUT_SIZE)(table, flat_idx)
    return out.reshape(B, _OUTPUT_SIZE, D)


# speedup vs baseline: 1.0640x; 1.0247x over previous
import functools

import jax
import jax.numpy as jnp
import numpy as np
from jax import lax
from jax.experimental import pallas as pl
from jax.experimental.pallas import tpu as pltpu
from jax.experimental.pallas import tpu_sc as plsc

_OUTPUT_SIZE = 1024


def _choose_indices(B, L):
    key = jax.random.key(42)
    noise = jax.random.uniform(jax.random.fold_in(key, 1), (B, L))
    indices = jnp.argsort(noise, axis=-1)[:, :_OUTPUT_SIZE]
    return jnp.sort(indices, axis=-1)


@functools.lru_cache(maxsize=None)
def _flat_indices_const(B, L):
    try:
        with jax.ensure_compile_time_eval():
            idx = _choose_indices(B, L)
            flat = (idx + jnp.arange(B)[:, None] * L).reshape(-1)
            return np.asarray(flat, dtype=np.int32)
    except Exception:
        return None


@functools.lru_cache(maxsize=None)
def _make_gather(V, D, B):
    info = plsc.get_sparse_core_info()
    NC, NS = info.num_cores, info.num_subcores
    NW = NC * NS
    assert B % NW == 0 and (B // NW) % 8 == 0
    b_per_w = B // NW
    chunk = min(8, b_per_w)
    n_chunks = b_per_w // chunk
    assert n_chunks % 2 == 0 and n_chunks >= 4
    mesh = plsc.VectorSubcoreMesh(core_axis_name="c", subcore_axis_name="s")

    @functools.partial(
        pl.kernel,
        mesh=mesh,
        out_type=jax.ShapeDtypeStruct((B, D), jnp.float32),
        scratch_types=[
            pltpu.VMEM((b_per_w,), jnp.int32),
        ]
        + [pltpu.VMEM((chunk, D), jnp.float32) for _ in range(6)]
        + [pltpu.SemaphoreType.DMA for _ in range(12)],
    )
    def gather_kernel(table_hbm, idx_hbm, out_hbm, idx_v, *scratch):
        bufs = scratch[:6]
        gsems = scratch[6:12]
        wsems = scratch[12:]
        wid = lax.axis_index("s") * NC + lax.axis_index("c")
        base = wid * b_per_w
        pltpu.sync_copy(idx_hbm.at[pl.ds(base, b_per_w)], idx_v)

        def gather_start(c, b):
            pltpu.async_copy(
                table_hbm.at[idx_v.at[pl.ds(c * chunk, chunk)]],
                bufs[b], gsems[b])

        def gather_wait(b):
            pltpu.make_async_copy(
                table_hbm.at[idx_v.at[pl.ds(0, chunk)]],
                bufs[b], gsems[b]).wait()

        def write_start(c, b):
            pltpu.async_copy(
                bufs[b], out_hbm.at[pl.ds(base + c * chunk, chunk)], wsems[b])

        def write_wait(b):
            pltpu.make_async_copy(
                bufs[b], out_hbm.at[pl.ds(base, chunk)], wsems[b]).wait()

        for p in range(5):
            gather_start(p, p)
        for c in range(n_chunks):
            b = c % 6
            gather_wait(b)
            write_start(c, b)
            nxt = c + 5
            if nxt < n_chunks:
                if nxt >= 6:
                    write_wait(nxt % 6)
                gather_start(nxt, nxt % 6)
        for c in range(n_chunks - 6, n_chunks):
            write_wait(c % 6)

    return gather_kernel


def kernel(inputs):
    B, L, D = inputs.shape
    const = _flat_indices_const(B, L)
    if const is not None:
        flat_idx = jnp.asarray(const)
    else:
        indices = _choose_indices(B, L)
        flat_idx = (indices + jnp.arange(B)[:, None] * L).reshape(-1).astype(jnp.int32)
    table = inputs.reshape(B * L, D)
    out = _make_gather(B * L, D, B * _OUTPUT_SIZE)(table, flat_idx)
    return out.reshape(B, _OUTPUT_SIZE, D)
